# baseline (speedup 1.0000x reference)
import functools

import jax
import jax.numpy as jnp
from jax import lax
from jax.experimental import pallas as pl
from jax.experimental.pallas import tpu as pltpu
from jax.experimental.pallas import tpu_sc as plsc

NUM_FREQ = 4
IN_DIM = 4
OUT_DIM = 4
NUM_BINS = 34
ROW = OUT_DIM * IN_DIM * NUM_FREQ
E = 800000

NC = 2
NS = 16
NW = NC * NS
L = 16

PER_W = E // NW
BE = 512
NFULL = PER_W // BE
TAIL = PER_W - NFULL * BE
DPAD = -(-TAIL // L) * L - TAIL
U = 4

_MAGIC = 8388608.0


def _bins_from_dists(d):
    x = jnp.clip((d - 2.4) / 0.4, 0.0, 33.0)
    r = (x + _MAGIC) - _MAGIC
    return r.astype(jnp.int32)


_mesh = plsc.VectorSubcoreMesh(core_axis_name="c", subcore_axis_name="s")


@functools.partial(
    pl.kernel,
    mesh=_mesh,
    out_type=jax.ShapeDtypeStruct((E, ROW), jnp.float32),
    scratch_types=[
        pltpu.VMEM((NUM_BINS * ROW,), jnp.float32),
        pltpu.VMEM((PER_W + DPAD,), jnp.float32),
        [pltpu.VMEM((BE, ROW), jnp.float32) for _ in range(2)],
        [pltpu.SemaphoreType.DMA for _ in range(2)],
    ],
    compiler_params=pltpu.CompilerParams(use_tc_tiling_on_sc=False,
                                         needs_layout_passes=False),
)
def _radial_sc(dists_hbm, table_hbm, out_hbm, tbl_v, d_v, rows, sem_w):
    wid = lax.axis_index("s") * NC + lax.axis_index("c")
    ebase = wid * PER_W

    d_v[pl.ds(PER_W + DPAD - L, L)] = jnp.zeros((L,), jnp.float32)
    pltpu.sync_copy(table_hbm, tbl_v)
    pltpu.sync_copy(dists_hbm.at[pl.ds(ebase, PER_W)], d_v.at[pl.ds(0, PER_W)])

    def drain(b, n_edges):
        pltpu.make_async_copy(rows[b].at[pl.ds(0, n_edges)],
                              out_hbm.at[pl.ds(ebase, n_edges)],
                              sem_w[b]).wait()

    def emit_group(base_e, off, b, nu):
        bvec = _bins_from_dists(d_v[pl.ds(off, L)]) * ROW
        for u in range(nu):
            e = base_e + u
            a = bvec[u]
            for k in range(ROW // L):
                rows[b][e, pl.ds(k * L, L)] = tbl_v[pl.ds(a + k * L, L)]

    def process(blk, b, n_edges):
        @pl.loop(0, n_edges // L)
        def _(g):
            emit_group(g * L, blk * BE + g * L, b, L)

        if n_edges % L:
            ngf = n_edges // L
            emit_group(ngf * L, blk * BE + ngf * L, b, n_edges % L)

        pltpu.async_copy(rows[b].at[pl.ds(0, n_edges)],
                         out_hbm.at[pl.ds(ebase + blk * BE, n_edges)],
                         sem_w[b])

    @pl.loop(0, NFULL, step=2)
    def _(j):
        for b in range(2):
            k = j + b

            @pl.when(k >= 2)
            def _():
                drain(b, BE)

            process(k, b, BE)

    drain(0, BE)
    process(NFULL, 0, TAIL)
    drain(1, BE)
    drain(0, TAIL)


def kernel(dists, bin_embedding):
    x = _radial_sc(dists.reshape(E), bin_embedding.reshape(NUM_BINS * ROW))
    return x.reshape(E, OUT_DIM, IN_DIM, NUM_FREQ)[:, :, None, :, None, :]
